# baseline (speedup 1.0000x reference)
import functools
import jax
import jax.numpy as jnp
from jax import lax
from jax.experimental import pallas as pl
from jax.experimental.pallas import tpu as pltpu
from jax.experimental.pallas import tpu_sc as plsc

_TC_ROWS = 512
_SC_CHUNK = 64


def _tc_body(tab_ref, out_ref):
    b = out_ref.shape[0]
    out_ref[...] = jnp.broadcast_to(tab_ref[...][None], (b,) + tab_ref.shape)


def _tc_broadcast(table, n, rows, embed):
    return pl.pallas_call(
        _tc_body,
        grid=(rows // _TC_ROWS,),
        in_specs=[pl.BlockSpec((_TC_ROWS, embed), lambda i: (i, 0))],
        out_specs=pl.BlockSpec((n, _TC_ROWS, embed), lambda i: (0, i, 0)),
        out_shape=jax.ShapeDtypeStruct((n, rows, embed), table.dtype),
    )(table)


def _sc_broadcast(table, n, rows, embed):
    info = plsc.get_sparse_core_info()
    nw = info.num_cores * info.num_subcores
    rows_per_w = rows // nw
    n_chunks = rows_per_w // _SC_CHUNK
    mesh = plsc.VectorSubcoreMesh(core_axis_name="c", subcore_axis_name="s")

    @functools.partial(
        pl.kernel,
        mesh=mesh,
        out_type=jax.ShapeDtypeStruct((n, rows, embed), table.dtype),
        scratch_types=[
            pltpu.VMEM((_SC_CHUNK, embed), table.dtype),
            pltpu.SemaphoreType.DMA,
        ],
    )
    def sc_kernel(tab_hbm, out_hbm, buf, sem):
        wid = lax.axis_index("s") * info.num_cores + lax.axis_index("c")
        base = wid * rows_per_w

        def chunk_body(c, carry):
            start = base + c * _SC_CHUNK
            pltpu.sync_copy(tab_hbm.at[pl.ds(start, _SC_CHUNK)], buf)
            handles = [
                pltpu.async_copy(buf, out_hbm.at[b, pl.ds(start, _SC_CHUNK)], sem)
                for b in range(n)
            ]
            for h in handles:
                h.wait()
            return carry

        lax.fori_loop(0, n_chunks, chunk_body, 0)

    return sc_kernel(table)


def kernel(src, tgt, src_pos_table, tgt_pos_table):
    n = src.shape[0]
    src_len = src.shape[1]
    tgt_len = tgt.shape[1]
    embed = src_pos_table.shape[1]

    tgt_out = _sc_broadcast(tgt_pos_table[:tgt_len], n, tgt_len, embed)
    src_out = _tc_broadcast(src_pos_table[:src_len], n, src_len, embed)
    return (src_out, tgt_out)


# speedup vs baseline: 1.0134x; 1.0134x over previous
import functools
import jax
import jax.numpy as jnp
from jax import lax
from jax.experimental import pallas as pl
from jax.experimental.pallas import tpu as pltpu
from jax.experimental.pallas import tpu_sc as plsc

_TC_ROWS = 512
_SC_CHUNK = 32


def _tc_body(tab_ref, out_ref):
    b = out_ref.shape[0]
    out_ref[...] = jnp.broadcast_to(tab_ref[...][None], (b,) + tab_ref.shape)


def _tc_broadcast(table, n, rows, embed):
    return pl.pallas_call(
        _tc_body,
        grid=(rows // _TC_ROWS,),
        in_specs=[pl.BlockSpec((_TC_ROWS, embed), lambda i: (i, 0))],
        out_specs=pl.BlockSpec((n, _TC_ROWS, embed), lambda i: (0, i, 0)),
        out_shape=jax.ShapeDtypeStruct((n, rows, embed), table.dtype),
    )(table)


def _sc_broadcast(table, n, rows, embed):
    info = plsc.get_sparse_core_info()
    nw = info.num_cores * info.num_subcores
    rows_per_w = rows // nw
    n_chunks = rows_per_w // _SC_CHUNK
    mesh = plsc.VectorSubcoreMesh(core_axis_name="c", subcore_axis_name="s")

    @functools.partial(
        pl.kernel,
        mesh=mesh,
        out_type=jax.ShapeDtypeStruct((n, rows, embed), table.dtype),
        scratch_types=[
            pltpu.VMEM((_SC_CHUNK, embed), table.dtype),
            pltpu.VMEM((_SC_CHUNK, embed), table.dtype),
            pltpu.SemaphoreType.DMA,
            pltpu.SemaphoreType.DMA,
            pltpu.SemaphoreType.DMA,
            pltpu.SemaphoreType.DMA,
        ],
    )
    def sc_kernel(tab_hbm, out_hbm, buf0, buf1, rsem0, rsem1, wsem0, wsem1):
        wid = lax.axis_index("s") * info.num_cores + lax.axis_index("c")
        base = wid * rows_per_w
        bufs = (buf0, buf1)
        rsems = (rsem0, rsem1)
        wsems = (wsem0, wsem1)

        reads = {}
        writes = {}

        def start_read(c):
            start = base + c * _SC_CHUNK
            reads[c] = pltpu.async_copy(
                tab_hbm.at[pl.ds(start, _SC_CHUNK)], bufs[c % 2], rsems[c % 2]
            )

        start_read(0)
        for c in range(n_chunks):
            if c >= 2:
                for h in writes[c - 2]:
                    h.wait()
            reads[c].wait()
            if c + 1 < n_chunks:
                if c >= 1:
                    for h in writes[c - 1]:
                        h.wait()
                    writes[c - 1] = []
                start_read(c + 1)
            start = base + c * _SC_CHUNK
            writes[c] = [
                pltpu.async_copy(
                    bufs[c % 2], out_hbm.at[b, pl.ds(start, _SC_CHUNK)], wsems[c % 2]
                )
                for b in range(n)
            ]
        for h in writes.get(n_chunks - 2, []):
            h.wait()
        for h in writes[n_chunks - 1]:
            h.wait()

    return sc_kernel(table)


def kernel(src, tgt, src_pos_table, tgt_pos_table):
    n = src.shape[0]
    src_len = src.shape[1]
    tgt_len = tgt.shape[1]
    embed = src_pos_table.shape[1]

    tgt_out = _sc_broadcast(tgt_pos_table[:tgt_len], n, tgt_len, embed)
    src_out = _tc_broadcast(src_pos_table[:src_len], n, src_len, embed)
    return (src_out, tgt_out)


# speedup vs baseline: 1.2305x; 1.2142x over previous
import jax
import jax.numpy as jnp
from jax.experimental import pallas as pl

_ROWS = 1024


def _bcast_body(tab_ref, out_ref):
    b = out_ref.shape[0]
    out_ref[...] = jnp.broadcast_to(tab_ref[...][None], (b,) + tab_ref.shape)


def _bcast(table, n, rows, embed):
    return pl.pallas_call(
        _bcast_body,
        grid=(rows // _ROWS,),
        in_specs=[pl.BlockSpec((_ROWS, embed), lambda i: (i, 0))],
        out_specs=pl.BlockSpec((n, _ROWS, embed), lambda i: (0, i, 0)),
        out_shape=jax.ShapeDtypeStruct((n, rows, embed), table.dtype),
    )(table)


def kernel(src, tgt, src_pos_table, tgt_pos_table):
    n = src.shape[0]
    src_len = src.shape[1]
    tgt_len = tgt.shape[1]
    embed = src_pos_table.shape[1]
    src_out = _bcast(src_pos_table[:src_len], n, src_len, embed)
    tgt_out = _bcast(tgt_pos_table[:tgt_len], n, tgt_len, embed)
    return (src_out, tgt_out)
